# baseline (speedup 1.0000x reference)
import functools

import jax
import jax.numpy as jnp
from jax import lax
from jax.experimental import pallas as pl
from jax.experimental.pallas import tpu as pltpu
from jax.experimental.pallas import tpu_sc as plsc

BATCH = 16384
HIST = 50
EMB = 32

NC = 2
NS = 16
NW = NC * NS
B_PER_W = BATCH // NW
NBUF = 3
LANES = 16
NBLK = B_PER_W // LANES

_mesh = plsc.VectorSubcoreMesh(core_axis_name="c", subcore_axis_name="s")


@functools.partial(
    pl.kernel,
    mesh=_mesh,
    out_type=jax.ShapeDtypeStruct((HIST, EMB, BATCH), jnp.float32),
    scratch_types=(
        [pltpu.VMEM((HIST, B_PER_W), jnp.int32)]
        + [pltpu.VMEM((B_PER_W, EMB), jnp.float32) for _ in range(NBUF)]
        + [pltpu.VMEM((EMB, B_PER_W + 1), jnp.float32) for _ in range(NBUF)]
        + [pltpu.SemaphoreType.DMA for _ in range(2 * NBUF)]
    ),
    compiler_params=pltpu.CompilerParams(
        use_tc_tiling_on_sc=False, needs_layout_passes=False
    ),
)
def _emb_gather(xt_hbm, table_hbm, out_hbm, idx_v, *rest):
    gbuf = rest[:NBUF]
    tbuf = rest[NBUF:2 * NBUF]
    gsem = rest[2 * NBUF:3 * NBUF]
    ssem = rest[3 * NBUF:]
    wid = lax.axis_index("s") * NC + lax.axis_index("c")
    b0 = wid * B_PER_W
    pltpu.sync_copy(xt_hbm.at[:, pl.ds(b0, B_PER_W)], idx_v)

    def start_gather(h, b):
        pltpu.async_copy(table_hbm.at[idx_v.at[h]], gbuf[b], gsem[b])

    def wait_gather(b):
        pltpu.make_async_copy(table_hbm.at[pl.ds(0, B_PER_W)], gbuf[b], gsem[b]).wait()

    def start_store(h, b):
        pltpu.async_copy(
            tbuf[b].at[:, pl.ds(0, B_PER_W)],
            out_hbm.at[h, :, pl.ds(b0, B_PER_W)],
            ssem[b],
        )

    def wait_store(b):
        pltpu.make_async_copy(
            tbuf[b].at[:, pl.ds(0, B_PER_W)],
            out_hbm.at[0, :, pl.ds(b0, B_PER_W)],
            ssem[b],
        ).wait()

    def transpose(b):
        iota = lax.broadcasted_iota(jnp.int32, (LANES,), 0)

        def blk_body(blk, carry):
            rbase = blk * LANES
            for half in range(EMB // LANES):
                c_vec = half * LANES + iota
                for k in range(LANES):
                    v = gbuf[b][rbase + k, pl.ds(half * LANES, LANES)]
                    r_vec = jnp.full((LANES,), rbase + k, jnp.int32)
                    plsc.store_scatter(tbuf[b], [c_vec, r_vec], v)
            return carry

        lax.fori_loop(0, NBLK, blk_body, 0)

    for b in range(NBUF):
        start_gather(b, b)

    def body(h2, carry):
        for b in range(NBUF):
            h = h2 * NBUF + b
            wait_gather(b)
            wait_store(b)
            transpose(b)
            start_store(h, b)

            @pl.when(h + NBUF < HIST)
            def _():
                start_gather(h + NBUF, b)
        return carry

    for b in range(NBUF):
        pltpu.async_copy(
            tbuf[b].at[:, pl.ds(0, B_PER_W)],
            out_hbm.at[HIST - 1, :, pl.ds(b0, B_PER_W)],
            ssem[b],
        )

    NGRP = HIST // NBUF
    lax.fori_loop(0, NGRP, body, 0)

    for b in range(HIST - NGRP * NBUF):
        h = NGRP * NBUF + b
        wait_gather(b)
        wait_store(b)
        transpose(b)
        start_store(h, b)
    for b in range(NBUF):
        wait_store(b)


def kernel(x, table):
    out_t = _emb_gather(x.T, table)
    return out_t.transpose(2, 0, 1)


# speedup vs baseline: 1.0065x; 1.0065x over previous
import functools

import jax
import jax.numpy as jnp
from jax import lax
from jax.experimental import pallas as pl
from jax.experimental.pallas import tpu as pltpu
from jax.experimental.pallas import tpu_sc as plsc

BATCH = 16384
HIST = 50
EMB = 32

NC = 2
NS = 16
NW = NC * NS
B_PER_W = BATCH // NW
NBUF = 2
LANES = 16
NBLK = B_PER_W // LANES

_mesh = plsc.VectorSubcoreMesh(core_axis_name="c", subcore_axis_name="s")


@functools.partial(
    pl.kernel,
    mesh=_mesh,
    out_type=jax.ShapeDtypeStruct((HIST, EMB, BATCH), jnp.float32),
    scratch_types=(
        [pltpu.VMEM((HIST, B_PER_W), jnp.int32)]
        + [pltpu.VMEM((B_PER_W, EMB), jnp.float32) for _ in range(NBUF)]
        + [pltpu.VMEM((EMB, B_PER_W + 1), jnp.float32) for _ in range(NBUF)]
        + [pltpu.SemaphoreType.DMA for _ in range(2 * NBUF)]
    ),
    compiler_params=pltpu.CompilerParams(
        use_tc_tiling_on_sc=False, needs_layout_passes=False
    ),
)
def _emb_gather(xt_hbm, table_hbm, out_hbm, idx_v, *rest):
    gbuf = rest[:NBUF]
    tbuf = rest[NBUF:2 * NBUF]
    gsem = rest[2 * NBUF:3 * NBUF]
    ssem = rest[3 * NBUF:]
    wid = lax.axis_index("s") * NC + lax.axis_index("c")
    b0 = wid * B_PER_W
    pltpu.sync_copy(xt_hbm.at[:, pl.ds(b0, B_PER_W)], idx_v)

    def start_gather(h, b):
        pltpu.async_copy(table_hbm.at[idx_v.at[h]], gbuf[b], gsem[b])

    def wait_gather(b):
        pltpu.make_async_copy(table_hbm.at[pl.ds(0, B_PER_W)], gbuf[b], gsem[b]).wait()

    def start_store(h, b):
        pltpu.async_copy(
            tbuf[b].at[:, pl.ds(0, B_PER_W)],
            out_hbm.at[h, :, pl.ds(b0, B_PER_W)],
            ssem[b],
        )

    def wait_store(b):
        pltpu.make_async_copy(
            tbuf[b].at[:, pl.ds(0, B_PER_W)],
            out_hbm.at[0, :, pl.ds(b0, B_PER_W)],
            ssem[b],
        ).wait()

    def transpose(b):
        iota = lax.broadcasted_iota(jnp.int32, (LANES,), 0)

        def blk_body(blk, carry):
            rbase = blk * LANES
            for half in range(EMB // LANES):
                c_vec = half * LANES + iota
                for k in range(LANES):
                    v = gbuf[b][rbase + k, pl.ds(half * LANES, LANES)]
                    r_vec = jnp.full((LANES,), rbase + k, jnp.int32)
                    plsc.store_scatter(tbuf[b], [c_vec, r_vec], v)
            return carry

        lax.fori_loop(0, NBLK, blk_body, 0)

    for b in range(NBUF):
        start_gather(b, b)

    def body(h2, carry):
        for b in range(NBUF):
            h = h2 * NBUF + b
            wait_gather(b)
            wait_store(b)
            transpose(b)
            start_store(h, b)
            start_gather(h + NBUF, b)
        return carry

    for b in range(NBUF):
        pltpu.async_copy(
            tbuf[b].at[:, pl.ds(0, B_PER_W)],
            out_hbm.at[HIST - 1, :, pl.ds(b0, B_PER_W)],
            ssem[b],
        )

    lax.fori_loop(0, HIST // NBUF - 1, body, 0)

    for b in range(NBUF):
        h = HIST - NBUF + b
        wait_gather(b)
        wait_store(b)
        transpose(b)
        start_store(h, b)
    for b in range(NBUF):
        wait_store(b)


def kernel(x, table):
    out_t = _emb_gather(x.T, table)
    return out_t.transpose(2, 0, 1)
